# baseline (speedup 1.0000x reference)
import jax
import jax.numpy as jnp
import numpy as np
from jax.experimental import pallas as pl
from jax.experimental.pallas import tpu as pltpu

ALPHA = 4


def _make_body(idx):
    def body(in_ref, slow_ref, fast_ref):
        fast_ref[...] = in_ref[...]
        for s, i in enumerate(idx):
            slow_ref[0, 0, s] = in_ref[0, 0, i]

    return body


def kernel(frames):
    B, C, T, H, W = frames.shape
    S = T // ALPHA
    idx = [int(v) for v in np.linspace(0.0, T - 1, S, dtype=np.float32).astype(np.int32)]
    slow, fast = pl.pallas_call(
        _make_body(idx),
        grid=(B, C, 2),
        in_specs=[
            pl.BlockSpec((1, 1, T, H // 2, W), lambda b, c, h: (b, c, 0, h, 0))
        ],
        out_specs=[
            pl.BlockSpec((1, 1, S, H // 2, W), lambda b, c, h: (b, c, 0, h, 0)),
            pl.BlockSpec((1, 1, T, H // 2, W), lambda b, c, h: (b, c, 0, h, 0)),
        ],
        out_shape=[
            jax.ShapeDtypeStruct((B, C, S, H, W), frames.dtype),
            jax.ShapeDtypeStruct((B, C, T, H, W), frames.dtype),
        ],
    )(frames)
    return slow, fast


# speedup vs baseline: 1.0156x; 1.0156x over previous
import jax
import jax.numpy as jnp
import numpy as np
from jax.experimental import pallas as pl
from jax.experimental.pallas import tpu as pltpu

ALPHA = 4


def _make_body(idx):
    def body(in_ref, slow_ref, fast_ref):
        fast_ref[...] = in_ref[...]
        for s, i in enumerate(idx):
            slow_ref[0, 0, s] = in_ref[0, 0, i]

    return body


def kernel(frames):
    B, C, T, H, W = frames.shape
    S = T // ALPHA
    idx = [int(v) for v in np.linspace(0.0, T - 1, S, dtype=np.float32).astype(np.int32)]
    slow, fast = pl.pallas_call(
        _make_body(idx),
        grid=(B, C),
        in_specs=[pl.BlockSpec((1, 1, T, H, W), lambda b, c: (b, c, 0, 0, 0))],
        out_specs=[
            pl.BlockSpec((1, 1, S, H, W), lambda b, c: (b, c, 0, 0, 0)),
            pl.BlockSpec((1, 1, T, H, W), lambda b, c: (b, c, 0, 0, 0)),
        ],
        out_shape=[
            jax.ShapeDtypeStruct((B, C, S, H, W), frames.dtype),
            jax.ShapeDtypeStruct((B, C, T, H, W), frames.dtype),
        ],
    )(frames)
    return slow, fast


# speedup vs baseline: 1.0161x; 1.0005x over previous
import jax
import numpy as np
from jax.experimental import pallas as pl

ALPHA = 4


def _make_body(idx):
    def body(in_ref, slow_ref, fast_ref):
        fast_ref[...] = in_ref[...]
        for s, i in enumerate(idx):
            slow_ref[0, 0, s] = in_ref[0, 0, i]

    return body


def kernel(frames):
    B, C, T, H, W = frames.shape
    S = T // ALPHA
    idx = [int(v) for v in np.linspace(0.0, T - 1, S, dtype=np.float32).astype(np.int32)]
    slow, fast = pl.pallas_call(
        _make_body(idx),
        grid=(B, C),
        in_specs=[pl.BlockSpec((1, 1, T, H, W), lambda b, c: (b, c, 0, 0, 0))],
        out_specs=[
            pl.BlockSpec((1, 1, S, H, W), lambda b, c: (b, c, 0, 0, 0)),
            pl.BlockSpec((1, 1, T, H, W), lambda b, c: (b, c, 0, 0, 0)),
        ],
        out_shape=[
            jax.ShapeDtypeStruct((B, C, S, H, W), frames.dtype),
            jax.ShapeDtypeStruct((B, C, T, H, W), frames.dtype),
        ],
    )(frames)
    return slow, fast
